# Initial kernel scaffold; baseline (speedup 1.0000x reference)
import jax
import jax.numpy as jnp
from jax.experimental import pallas as pl


def kernel(x_A, x_B, ln_gA, ln_bA, resW_A, W1_A, b1_A, W2_A, b2_A, ln_gB, ln_bB, resW_B, W1_B, b1_B, W2_B, b2_B, relW_AB, gate_AB, relW_BA, gate_BA, edge_index_A_B, edge_index_B_A):
    raise NotImplementedError("write your pallas kernel here")


# speedup vs baseline: 3.6348x; 3.6348x over previous
import functools

import jax
import jax.numpy as jnp
from jax import lax
from jax.experimental import pallas as pl
from jax.experimental.pallas import tpu as pltpu
from jax.experimental.pallas import tpu_sc as plsc

N = 10000
E = 160000
D = 256
DH = 128
HID = 512
NT = 16
CH = 80
EPT = E // NT
NCH = EPT // CH
NPAD = 10240
RPT = NPAD // NT

_f32 = jnp.float32


def _sc_mesh():
    return plsc.VectorSubcoreMesh(core_axis_name="c", subcore_axis_name="s")


def _deg_body(src1, dst1, src2, dst2, z1,
              dS1, dD1, dS2, dD2,
              accS, accD, srcv, dstv, ones_v):
    c = lax.axis_index("c")
    s = lax.axis_index("s")
    for i in range(CH // 16):
        ones_v[pl.ds(i * 16, 16)] = jnp.ones((16,), _f32)
    pltpu.sync_copy(z1, accS.at[pl.ds(s * RPT, RPT)])
    pltpu.sync_copy(z1, accD.at[pl.ds(s * RPT, RPT)])
    plsc.subcore_barrier()

    def accum(src3d, dst3d):
        pltpu.sync_copy(src3d.at[s], srcv)
        pltpu.sync_copy(dst3d.at[s], dstv)

        def body(j, carry):
            pltpu.sync_copy(ones_v, accS.at[srcv.at[j]], add=True)
            pltpu.sync_copy(ones_v, accD.at[dstv.at[j]], add=True)
            return carry

        lax.fori_loop(0, NCH, body, 0)

    @pl.when(c == 0)
    def _():
        accum(src1, dst1)

    @pl.when(c == 1)
    def _():
        accum(src2, dst2)

    plsc.subcore_barrier()

    @pl.when(c == 0)
    def _():
        pltpu.sync_copy(accS.at[pl.ds(s * RPT, RPT)], dS1.at[pl.ds(s * RPT, RPT)])
        pltpu.sync_copy(accD.at[pl.ds(s * RPT, RPT)], dD1.at[pl.ds(s * RPT, RPT)])

    @pl.when(c == 1)
    def _():
        pltpu.sync_copy(accS.at[pl.ds(s * RPT, RPT)], dS2.at[pl.ds(s * RPT, RPT)])
        pltpu.sync_copy(accD.at[pl.ds(s * RPT, RPT)], dD2.at[pl.ds(s * RPT, RPT)])


def _degrees(src1, dst1, src2, dst2):
    z1 = jnp.zeros((RPT,), _f32)
    out = jax.ShapeDtypeStruct((NPAD,), _f32)
    fn = pl.kernel(
        _deg_body,
        out_type=[out, out, out, out],
        mesh=_sc_mesh(),
        scratch_types=[
            pltpu.VMEM_SHARED((NPAD,), _f32),
            pltpu.VMEM_SHARED((NPAD,), _f32),
            pltpu.VMEM((NCH, CH), jnp.int32),
            pltpu.VMEM((NCH, CH), jnp.int32),
            pltpu.VMEM((CH,), _f32),
        ],
    )
    return fn(src1, dst1, src2, dst2, z1)


def _agg_body(xAlo, xAhi, xBlo, xBhi, src1, dst1, src2, dst2, z2,
              oBlo, oBhi, oAlo, oAhi,
              acc, srcv, dstv, bufA, bufB, semA, semB):
    c = lax.axis_index("c")
    s = lax.axis_index("s")

    def aggregate(table, src2d, dst3d, out):
        pltpu.sync_copy(z2, acc.at[pl.ds(s * RPT, RPT)])
        pltpu.sync_copy(src2d.at[s], srcv)
        pltpu.sync_copy(dst3d.at[s], dstv)
        plsc.subcore_barrier()

        def sidx(j):
            return srcv.at[pl.ds(j * CH, CH)]

        pltpu.async_copy(table.at[sidx(0)], bufA, semA)

        def body(i, carry):
            ja = 2 * i
            jb = 2 * i + 1
            pltpu.make_async_copy(table.at[sidx(ja)], bufA, semA).wait()
            pltpu.async_copy(table.at[sidx(jb)], bufB, semB)
            pltpu.sync_copy(bufA, acc.at[dstv.at[ja]], add=True)
            pltpu.make_async_copy(table.at[sidx(jb)], bufB, semB).wait()
            pltpu.async_copy(table.at[sidx(ja + 2)], bufA, semA)
            pltpu.sync_copy(bufB, acc.at[dstv.at[jb]], add=True)
            return carry

        lax.fori_loop(0, (NCH - 1) // 2, body, 0)
        pltpu.make_async_copy(table.at[sidx(NCH - 1)], bufA, semA).wait()
        pltpu.sync_copy(bufA, acc.at[dstv.at[NCH - 1]], add=True)
        plsc.subcore_barrier()
        pltpu.sync_copy(acc.at[pl.ds(s * RPT, RPT)], out.at[pl.ds(s * RPT, RPT)])
        plsc.subcore_barrier()

    @pl.when(c == 0)
    def _():
        aggregate(xAlo, src1, dst1, oBlo)
        aggregate(xBlo, src2, dst2, oAlo)

    @pl.when(c == 1)
    def _():
        aggregate(xAhi, src1, dst1, oBhi)
        aggregate(xBhi, src2, dst2, oAhi)


def _aggregate(xAlo, xAhi, xBlo, xBhi, src1, dst1, src2, dst2):
    z2 = jnp.zeros((RPT, DH), _f32)
    out = jax.ShapeDtypeStruct((NPAD, DH), _f32)
    fn = pl.kernel(
        _agg_body,
        out_type=[out, out, out, out],
        mesh=_sc_mesh(),
        scratch_types=[
            pltpu.VMEM_SHARED((NPAD, DH), _f32),
            pltpu.VMEM((EPT,), jnp.int32),
            pltpu.VMEM((NCH, CH), jnp.int32),
            pltpu.VMEM((CH, DH), _f32),
            pltpu.VMEM((CH, DH), _f32),
            pltpu.SemaphoreType.DMA,
            pltpu.SemaphoreType.DMA,
        ],
    )
    return fn(xAlo, xAhi, xBlo, xBhi, src1, dst1, src2, dst2, z2)


BR = 1000


def _prep_body(x_ref, g_ref, b_ref, w_ref, deg_ref, proj_ref, lo_ref, hi_ref):
    x = x_ref[...]
    m = jnp.mean(x, axis=-1, keepdims=True)
    xc = x - m
    v = jnp.mean(xc * xc, axis=-1, keepdims=True)
    ln = xc * lax.rsqrt(v + 1e-5) * g_ref[...] + b_ref[...]
    proj_ref[...] = jnp.dot(ln, w_ref[...], preferred_element_type=_f32)
    deg = deg_ref[...]
    ds = jnp.where(deg > 0, lax.rsqrt(jnp.where(deg > 0, deg, 1.0)), 0.0)
    xs = x * ds
    lo_ref[...] = xs[:, :DH]
    hi_ref[...] = xs[:, DH:]


def _prep(x, g, b, w, deg):
    nb = N // BR
    return pl.pallas_call(
        _prep_body,
        grid=(nb,),
        in_specs=[
            pl.BlockSpec((BR, D), lambda i: (i, 0)),
            pl.BlockSpec((1, D), lambda i: (0, 0)),
            pl.BlockSpec((1, D), lambda i: (0, 0)),
            pl.BlockSpec((D, D), lambda i: (0, 0)),
            pl.BlockSpec((BR, 1), lambda i: (i, 0)),
        ],
        out_specs=[
            pl.BlockSpec((BR, D), lambda i: (i, 0)),
            pl.BlockSpec((BR, DH), lambda i: (i, 0)),
            pl.BlockSpec((BR, DH), lambda i: (i, 0)),
        ],
        out_shape=[
            jax.ShapeDtypeStruct((N, D), _f32),
            jax.ShapeDtypeStruct((N, DH), _f32),
            jax.ShapeDtypeStruct((N, DH), _f32),
        ],
    )(x, g.reshape(1, D), b.reshape(1, D), w, deg)


def _erf(x):
    a1, a2, a3, a4, a5 = (0.254829592, -0.284496736, 1.421413741,
                          -1.453152027, 1.061405429)
    p = 0.3275911
    ax = jnp.abs(x)
    t = 1.0 / (1.0 + p * ax)
    poly = ((((a5 * t + a4) * t + a3) * t + a2) * t + a1) * t
    y = 1.0 - poly * jnp.exp(-ax * ax)
    return jnp.sign(x) * y


def _gelu(u):
    return 0.5 * u * (1.0 + _erf(u * 0.7071067811865476))


def _final_body(proj_ref, lo_ref, hi_ref, deg_ref, relW_ref, gate_ref,
                W1_ref, b1_ref, W2_ref, b2_ref, out_ref):
    proj = proj_ref[...]
    deg = deg_ref[...]
    ds = jnp.where(deg > 0, lax.rsqrt(jnp.where(deg > 0, deg, 1.0)), 0.0)
    gate = gate_ref[0, 0]
    mlo = lo_ref[...] * ds
    mhi = hi_ref[...] * ds
    msg = (jnp.dot(mlo, relW_ref[:DH, :], preferred_element_type=_f32)
           + jnp.dot(mhi, relW_ref[DH:, :], preferred_element_type=_f32)) * gate
    u1 = (jnp.dot(proj, W1_ref[:D, :], preferred_element_type=_f32)
          + jnp.dot(msg, W1_ref[D:, :], preferred_element_type=_f32)
          + b1_ref[...])
    h = _gelu(u1)
    out_ref[...] = proj + jnp.dot(h, W2_ref[...], preferred_element_type=_f32) + b2_ref[...]


def _final(proj, lo, hi, deg, relW, gate, W1, b1, W2, b2):
    nb = N // BR
    return pl.pallas_call(
        _final_body,
        grid=(nb,),
        in_specs=[
            pl.BlockSpec((BR, D), lambda i: (i, 0)),
            pl.BlockSpec((BR, DH), lambda i: (i, 0)),
            pl.BlockSpec((BR, DH), lambda i: (i, 0)),
            pl.BlockSpec((BR, 1), lambda i: (i, 0)),
            pl.BlockSpec((D, D), lambda i: (0, 0)),
            pl.BlockSpec((1, 1), lambda i: (0, 0)),
            pl.BlockSpec((2 * D, HID), lambda i: (0, 0)),
            pl.BlockSpec((1, HID), lambda i: (0, 0)),
            pl.BlockSpec((HID, D), lambda i: (0, 0)),
            pl.BlockSpec((1, D), lambda i: (0, 0)),
        ],
        out_specs=pl.BlockSpec((BR, D), lambda i: (i, 0)),
        out_shape=jax.ShapeDtypeStruct((N, D), _f32),
    )(proj, lo, hi, deg, relW, gate.reshape(1, 1), W1, b1.reshape(1, HID),
      W2, b2.reshape(1, D))


def kernel(x_A, x_B, ln_gA, ln_bA, resW_A, W1_A, b1_A, W2_A, b2_A,
           ln_gB, ln_bB, resW_B, W1_B, b1_B, W2_B, b2_B,
           relW_AB, gate_AB, relW_BA, gate_BA,
           edge_index_A_B, edge_index_B_A):
    src1 = edge_index_A_B[0].astype(jnp.int32).reshape(NT, NCH, CH)
    dst1 = edge_index_A_B[1].astype(jnp.int32).reshape(NT, NCH, CH)
    src2 = edge_index_B_A[0].astype(jnp.int32).reshape(NT, NCH, CH)
    dst2 = edge_index_B_A[1].astype(jnp.int32).reshape(NT, NCH, CH)

    srcF1 = src1.reshape(NT, EPT)
    srcF2 = src2.reshape(NT, EPT)

    dS1, dD1, dS2, dD2 = _degrees(src1, dst1, src2, dst2)

    projA, loA, hiA = _prep(x_A, ln_gA, ln_bA, resW_A, dS1[:N].reshape(N, 1))
    projB, loB, hiB = _prep(x_B, ln_gB, ln_bB, resW_B, dS2[:N].reshape(N, 1))

    aggBlo, aggBhi, aggAlo, aggAhi = _aggregate(
        loA, hiA, loB, hiB, srcF1, dst1, srcF2, dst2)

    outA = _final(projA, aggAlo[:N], aggAhi[:N], dD2[:N].reshape(N, 1),
                  relW_BA, gate_BA, W1_A, b1_A, W2_A, b2_A)
    outB = _final(projB, aggBlo[:N], aggBhi[:N], dD1[:N].reshape(N, 1),
                  relW_AB, gate_AB, W1_B, b1_B, W2_B, b2_B)
    return jnp.concatenate([outA, outB], axis=0)


# speedup vs baseline: 4.5048x; 1.2394x over previous
import functools

import jax
import jax.numpy as jnp
from jax import lax
from jax.experimental import pallas as pl
from jax.experimental.pallas import tpu as pltpu
from jax.experimental.pallas import tpu_sc as plsc

N = 10000
E = 160000
D = 256
DH = 128
HID = 512
NT = 16
CH = 80
EPT = E // NT
NCH = EPT // CH
NPAD = 10240
RPT = NPAD // NT

_f32 = jnp.float32


def _sc_mesh():
    return plsc.VectorSubcoreMesh(core_axis_name="c", subcore_axis_name="s")


def _deg_body(src1, dst1, src2, dst2, z1,
              dS1, dD1, dS2, dD2,
              accS, accD, srcv, dstv, ones_v):
    c = lax.axis_index("c")
    s = lax.axis_index("s")
    for i in range(CH // 16):
        ones_v[pl.ds(i * 16, 16)] = jnp.ones((16,), _f32)
    pltpu.sync_copy(z1, accS.at[pl.ds(s * RPT, RPT)])
    pltpu.sync_copy(z1, accD.at[pl.ds(s * RPT, RPT)])
    plsc.subcore_barrier()

    def accum(src3d, dst3d):
        pltpu.sync_copy(src3d.at[s], srcv)
        pltpu.sync_copy(dst3d.at[s], dstv)

        def body(j, carry):
            pltpu.sync_copy(ones_v, accS.at[srcv.at[j]], add=True)
            pltpu.sync_copy(ones_v, accD.at[dstv.at[j]], add=True)
            return carry

        lax.fori_loop(0, NCH, body, 0)

    @pl.when(c == 0)
    def _():
        accum(src1, dst1)

    @pl.when(c == 1)
    def _():
        accum(src2, dst2)

    plsc.subcore_barrier()

    @pl.when(c == 0)
    def _():
        pltpu.sync_copy(accS.at[pl.ds(s * RPT, RPT)], dS1.at[pl.ds(s * RPT, RPT)])
        pltpu.sync_copy(accD.at[pl.ds(s * RPT, RPT)], dD1.at[pl.ds(s * RPT, RPT)])

    @pl.when(c == 1)
    def _():
        pltpu.sync_copy(accS.at[pl.ds(s * RPT, RPT)], dS2.at[pl.ds(s * RPT, RPT)])
        pltpu.sync_copy(accD.at[pl.ds(s * RPT, RPT)], dD2.at[pl.ds(s * RPT, RPT)])


def _degrees(src1, dst1, src2, dst2):
    z1 = jnp.zeros((RPT,), _f32)
    out = jax.ShapeDtypeStruct((NPAD,), _f32)
    fn = pl.kernel(
        _deg_body,
        out_type=[out, out, out, out],
        mesh=_sc_mesh(),
        scratch_types=[
            pltpu.VMEM_SHARED((NPAD,), _f32),
            pltpu.VMEM_SHARED((NPAD,), _f32),
            pltpu.VMEM((NCH, CH), jnp.int32),
            pltpu.VMEM((NCH, CH), jnp.int32),
            pltpu.VMEM((CH,), _f32),
        ],
    )
    return fn(src1, dst1, src2, dst2, z1)


CH2 = 128
NCH2 = EPT // CH2
TAIL = EPT - NCH2 * CH2


def _agg_body(xAlo, xAhi, xBlo, xBhi, src1, dst1, src2, dst2, z2,
              oBlo, oBhi, oAlo, oAhi,
              acc, sb0, sb1, sb2, db0, db1, db2, rb0, rb1, tsb, tdb, trb,
              is0, is1, is2, gs0, gs1, ss0, ss1):
    c = lax.axis_index("c")
    s = lax.axis_index("s")
    sbufs = (sb0, sb1, sb2)
    dbufs = (db0, db1, db2)
    rbufs = (rb0, rb1)
    isems = (is0, is1, is2)
    gsems = (gs0, gs1)
    ssems = (ss0, ss1)
    base = s * EPT

    def aggregate(table, srcE, dstE, out):
        pltpu.sync_copy(z2, acc.at[pl.ds(s * RPT, RPT)])
        plsc.subcore_barrier()

        def fire_idx(cn, k):
            pltpu.async_copy(srcE.at[pl.ds(base + cn * CH2, CH2)], sbufs[k], isems[k])
            pltpu.async_copy(dstE.at[pl.ds(base + cn * CH2, CH2)], dbufs[k], isems[k])

        def wait_idx(cn, k):
            pltpu.make_async_copy(srcE.at[pl.ds(base + cn * CH2, CH2)], sbufs[k], isems[k]).wait()
            pltpu.make_async_copy(dstE.at[pl.ds(base + cn * CH2, CH2)], dbufs[k], isems[k]).wait()

        def fire_gather(k, r):
            pltpu.async_copy(table.at[sbufs[k]], rbufs[r], gsems[r])

        def wait_gather(k, r):
            pltpu.make_async_copy(table.at[sbufs[k]], rbufs[r], gsems[r]).wait()

        def fire_scatter(k, r):
            pltpu.async_copy(rbufs[r], acc.at[dbufs[k]], ssems[r], add=True)

        def wait_scatter(k, r):
            pltpu.make_async_copy(rbufs[r], acc.at[dbufs[k]], ssems[r]).wait()

        def full_step(cn, u):
            wait_scatter((u - 1) % 3, (u - 1) % 2)
            fire_idx(cn + 2, (u + 2) % 3)
            wait_idx(cn + 1, (u + 1) % 3)
            fire_gather((u + 1) % 3, (u + 1) % 2)
            wait_gather(u % 3, u % 2)
            fire_scatter(u % 3, u % 2)

        fire_idx(0, 0)
        fire_idx(1, 1)
        wait_idx(0, 0)
        fire_gather(0, 0)
        fire_idx(2, 2)
        wait_idx(1, 1)
        fire_gather(1, 1)
        wait_gather(0, 0)
        fire_scatter(0, 0)

        def group(g, carry):
            cbase = 1 + 6 * g
            for uu in range(6):
                full_step(cbase + uu, (1 + uu) % 6)
            return carry

        lax.fori_loop(0, 12, group, 0)
        for cn in (73, 74, 75):
            full_step(cn, cn % 6)

        wait_scatter(75 % 3, 75 % 2)
        wait_idx(NCH2 - 1, (NCH2 - 1) % 3)
        fire_gather((NCH2 - 1) % 3, (NCH2 - 1) % 2)
        wait_gather(76 % 3, 76 % 2)
        fire_scatter(76 % 3, 76 % 2)
        wait_gather((NCH2 - 1) % 3, (NCH2 - 1) % 2)
        fire_scatter((NCH2 - 1) % 3, (NCH2 - 1) % 2)
        wait_scatter(76 % 3, 76 % 2)
        wait_scatter((NCH2 - 1) % 3, (NCH2 - 1) % 2)

        pltpu.sync_copy(srcE.at[pl.ds(base + NCH2 * CH2, TAIL)], tsb)
        pltpu.sync_copy(dstE.at[pl.ds(base + NCH2 * CH2, TAIL)], tdb)
        pltpu.async_copy(table.at[tsb], trb, is0).wait()
        pltpu.sync_copy(trb, acc.at[tdb], add=True)

        plsc.subcore_barrier()
        pltpu.sync_copy(acc.at[pl.ds(s * RPT, RPT)], out.at[pl.ds(s * RPT, RPT)])
        plsc.subcore_barrier()

    @pl.when(c == 0)
    def _():
        aggregate(xAlo, src1, dst1, oBlo)
        aggregate(xBlo, src2, dst2, oAlo)

    @pl.when(c == 1)
    def _():
        aggregate(xAhi, src1, dst1, oBhi)
        aggregate(xBhi, src2, dst2, oAhi)


def _aggregate(xAlo, xAhi, xBlo, xBhi, src1, dst1, src2, dst2):
    z2 = jnp.zeros((RPT, DH), _f32)
    out = jax.ShapeDtypeStruct((NPAD, DH), _f32)
    idxbuf = pltpu.VMEM((CH2,), jnp.int32)
    fn = pl.kernel(
        _agg_body,
        out_type=[out, out, out, out],
        mesh=_sc_mesh(),
        scratch_types=[
            pltpu.VMEM_SHARED((NPAD, DH), _f32),
            idxbuf, idxbuf, idxbuf,
            idxbuf, idxbuf, idxbuf,
            pltpu.VMEM((CH2, DH), _f32),
            pltpu.VMEM((CH2, DH), _f32),
            pltpu.VMEM((TAIL,), jnp.int32),
            pltpu.VMEM((TAIL,), jnp.int32),
            pltpu.VMEM((TAIL, DH), _f32),
            pltpu.SemaphoreType.DMA,
            pltpu.SemaphoreType.DMA,
            pltpu.SemaphoreType.DMA,
            pltpu.SemaphoreType.DMA,
            pltpu.SemaphoreType.DMA,
            pltpu.SemaphoreType.DMA,
            pltpu.SemaphoreType.DMA,
        ],
    )
    return fn(xAlo, xAhi, xBlo, xBhi, src1, dst1, src2, dst2, z2)


BR = 1000


def _prep_body(x_ref, g_ref, b_ref, w_ref, deg_ref, proj_ref, lo_ref, hi_ref):
    x = x_ref[...]
    m = jnp.mean(x, axis=-1, keepdims=True)
    xc = x - m
    v = jnp.mean(xc * xc, axis=-1, keepdims=True)
    ln = xc * lax.rsqrt(v + 1e-5) * g_ref[...] + b_ref[...]
    bf = jnp.bfloat16
    proj_ref[...] = jnp.dot(ln.astype(bf), w_ref[...].astype(bf),
                            preferred_element_type=_f32)
    deg = deg_ref[...]
    ds = jnp.where(deg > 0, lax.rsqrt(jnp.where(deg > 0, deg, 1.0)), 0.0)
    xs = x * ds
    lo_ref[...] = xs[:, :DH]
    hi_ref[...] = xs[:, DH:]


def _prep(x, g, b, w, deg):
    nb = N // BR
    return pl.pallas_call(
        _prep_body,
        grid=(nb,),
        in_specs=[
            pl.BlockSpec((BR, D), lambda i: (i, 0)),
            pl.BlockSpec((1, D), lambda i: (0, 0)),
            pl.BlockSpec((1, D), lambda i: (0, 0)),
            pl.BlockSpec((D, D), lambda i: (0, 0)),
            pl.BlockSpec((BR, 1), lambda i: (i, 0)),
        ],
        out_specs=[
            pl.BlockSpec((BR, D), lambda i: (i, 0)),
            pl.BlockSpec((BR, DH), lambda i: (i, 0)),
            pl.BlockSpec((BR, DH), lambda i: (i, 0)),
        ],
        out_shape=[
            jax.ShapeDtypeStruct((N, D), _f32),
            jax.ShapeDtypeStruct((N, DH), _f32),
            jax.ShapeDtypeStruct((N, DH), _f32),
        ],
    )(x, g.reshape(1, D), b.reshape(1, D), w, deg)


def _erf(x):
    a1, a2, a3, a4, a5 = (0.254829592, -0.284496736, 1.421413741,
                          -1.453152027, 1.061405429)
    p = 0.3275911
    ax = jnp.abs(x)
    t = 1.0 / (1.0 + p * ax)
    poly = ((((a5 * t + a4) * t + a3) * t + a2) * t + a1) * t
    y = 1.0 - poly * jnp.exp(-ax * ax)
    return jnp.sign(x) * y


def _gelu(u):
    return 0.5 * u * (1.0 + _erf(u * 0.7071067811865476))


def _final_body(proj_ref, lo_ref, hi_ref, deg_ref, relW_ref, gate_ref,
                W1_ref, b1_ref, W2_ref, b2_ref, out_ref):
    bf = jnp.bfloat16
    proj = proj_ref[...]
    deg = deg_ref[...]
    ds = jnp.where(deg > 0, lax.rsqrt(jnp.where(deg > 0, deg, 1.0)), 0.0)
    gate = gate_ref[0, 0]
    mlo = (lo_ref[...] * ds).astype(bf)
    mhi = (hi_ref[...] * ds).astype(bf)
    msg = (jnp.dot(mlo, relW_ref[:DH, :].astype(bf), preferred_element_type=_f32)
           + jnp.dot(mhi, relW_ref[DH:, :].astype(bf), preferred_element_type=_f32)) * gate
    u1 = (jnp.dot(proj.astype(bf), W1_ref[:D, :].astype(bf), preferred_element_type=_f32)
          + jnp.dot(msg.astype(bf), W1_ref[D:, :].astype(bf), preferred_element_type=_f32)
          + b1_ref[...])
    h = _gelu(u1)
    out_ref[...] = (proj
                    + jnp.dot(h.astype(bf), W2_ref[...].astype(bf),
                              preferred_element_type=_f32)
                    + b2_ref[...])


def _final(proj, lo, hi, deg, relW, gate, W1, b1, W2, b2):
    nb = N // BR
    return pl.pallas_call(
        _final_body,
        grid=(nb,),
        in_specs=[
            pl.BlockSpec((BR, D), lambda i: (i, 0)),
            pl.BlockSpec((BR, DH), lambda i: (i, 0)),
            pl.BlockSpec((BR, DH), lambda i: (i, 0)),
            pl.BlockSpec((BR, 1), lambda i: (i, 0)),
            pl.BlockSpec((D, D), lambda i: (0, 0)),
            pl.BlockSpec((1, 1), lambda i: (0, 0)),
            pl.BlockSpec((2 * D, HID), lambda i: (0, 0)),
            pl.BlockSpec((1, HID), lambda i: (0, 0)),
            pl.BlockSpec((HID, D), lambda i: (0, 0)),
            pl.BlockSpec((1, D), lambda i: (0, 0)),
        ],
        out_specs=pl.BlockSpec((BR, D), lambda i: (i, 0)),
        out_shape=jax.ShapeDtypeStruct((N, D), _f32),
    )(proj, lo, hi, deg, relW, gate.reshape(1, 1), W1, b1.reshape(1, HID),
      W2, b2.reshape(1, D))


def kernel(x_A, x_B, ln_gA, ln_bA, resW_A, W1_A, b1_A, W2_A, b2_A,
           ln_gB, ln_bB, resW_B, W1_B, b1_B, W2_B, b2_B,
           relW_AB, gate_AB, relW_BA, gate_BA,
           edge_index_A_B, edge_index_B_A):
    src1 = edge_index_A_B[0].astype(jnp.int32).reshape(NT, NCH, CH)
    dst1 = edge_index_A_B[1].astype(jnp.int32).reshape(NT, NCH, CH)
    src2 = edge_index_B_A[0].astype(jnp.int32).reshape(NT, NCH, CH)
    dst2 = edge_index_B_A[1].astype(jnp.int32).reshape(NT, NCH, CH)

    srcF1 = src1.reshape(E)
    dstF1 = dst1.reshape(E)
    srcF2 = src2.reshape(E)
    dstF2 = dst2.reshape(E)

    dS1, dD1, dS2, dD2 = _degrees(src1, dst1, src2, dst2)

    projA, loA, hiA = _prep(x_A, ln_gA, ln_bA, resW_A, dS1[:N].reshape(N, 1))
    projB, loB, hiB = _prep(x_B, ln_gB, ln_bB, resW_B, dS2[:N].reshape(N, 1))

    aggBlo, aggBhi, aggAlo, aggAhi = _aggregate(
        loA, hiA, loB, hiB, srcF1, dstF1, srcF2, dstF2)

    outA = _final(projA, aggAlo[:N], aggAhi[:N], dD2[:N].reshape(N, 1),
                  relW_BA, gate_BA, W1_A, b1_A, W2_A, b2_A)
    outB = _final(projB, aggBlo[:N], aggBhi[:N], dD1[:N].reshape(N, 1),
                  relW_AB, gate_AB, W1_B, b1_B, W2_B, b2_B)
    return jnp.concatenate([outA, outB], axis=0)


# speedup vs baseline: 4.9887x; 1.1074x over previous
import jax
import jax.numpy as jnp
from jax import lax
from jax.experimental import pallas as pl
from jax.experimental.pallas import tpu as pltpu
from jax.experimental.pallas import tpu_sc as plsc

N = 10000
E = 160000
D = 256
DH = 128
HID = 512
NT = 16
EPT = E // NT
NPAD = 10240
RPT = NPAD // NT

CH1 = 80
NCH1 = EPT // CH1
KB = 25

CH2 = 128
NCH2 = EPT // CH2
TAIL = EPT - NCH2 * CH2

_f32 = jnp.float32
_bf = jnp.bfloat16


def _sc_mesh():
    return plsc.VectorSubcoreMesh(core_axis_name="c", subcore_axis_name="s")


def _deg_body(src1, dst1, src2, dst2, z1,
              degS, degD,
              accS, accD, srcv, dstv, ones_v, semS, semD):
    c = lax.axis_index("c")
    s = lax.axis_index("s")
    for i in range(CH1 // 16):
        ones_v[pl.ds(i * 16, 16)] = jnp.ones((16,), _f32)
    pltpu.sync_copy(z1, accS.at[pl.ds(s * RPT, RPT)])
    pltpu.sync_copy(z1, accD.at[pl.ds(s * RPT, RPT)])
    plsc.subcore_barrier()

    def accum(src3d, dst3d):
        pltpu.sync_copy(src3d.at[s], srcv)
        pltpu.sync_copy(dst3d.at[s], dstv)

        def group(g, carry):
            for u in range(KB):
                j = KB * g + u
                pltpu.async_copy(ones_v, accS.at[srcv.at[j]], semS, add=True)
                pltpu.async_copy(ones_v, accD.at[dstv.at[j]], semD, add=True)
            for u in range(KB):
                j = KB * g + u
                pltpu.make_async_copy(ones_v, accS.at[srcv.at[j]], semS).wait()
                pltpu.make_async_copy(ones_v, accD.at[dstv.at[j]], semD).wait()
            return carry

        lax.fori_loop(0, NCH1 // KB, group, 0)

    @pl.when(c == 0)
    def _():
        accum(src1, dst1)

    @pl.when(c == 1)
    def _():
        accum(src2, dst2)

    plsc.subcore_barrier()
    @pl.when(c == 0)
    def _():
        pltpu.sync_copy(accS.at[pl.ds(s * RPT, RPT)], degS.at[0, pl.ds(s * RPT, RPT)])
        pltpu.sync_copy(accD.at[pl.ds(s * RPT, RPT)], degD.at[1, pl.ds(s * RPT, RPT)])

    @pl.when(c == 1)
    def _():
        pltpu.sync_copy(accS.at[pl.ds(s * RPT, RPT)], degS.at[1, pl.ds(s * RPT, RPT)])
        pltpu.sync_copy(accD.at[pl.ds(s * RPT, RPT)], degD.at[0, pl.ds(s * RPT, RPT)])


def _degrees(src1, dst1, src2, dst2):
    z1 = jnp.zeros((RPT,), _f32)
    out = jax.ShapeDtypeStruct((2, NPAD), _f32)
    fn = pl.kernel(
        _deg_body,
        out_type=[out, out],
        mesh=_sc_mesh(),
        scratch_types=[
            pltpu.VMEM_SHARED((NPAD,), _f32),
            pltpu.VMEM_SHARED((NPAD,), _f32),
            pltpu.VMEM((NCH1, CH1), jnp.int32),
            pltpu.VMEM((NCH1, CH1), jnp.int32),
            pltpu.VMEM((CH1,), _f32),
            pltpu.SemaphoreType.DMA,
            pltpu.SemaphoreType.DMA,
        ],
    )
    return fn(src1, dst1, src2, dst2, z1)


def _agg_body(tab_lo, tab_hi, src1, dst1, src2, dst2, z2,
              out_lo, out_hi,
              acc, sb0, sb1, sb2, db0, db1, db2, rb0, rb1, tsb, tdb, trb,
              is0, is1, is2, gs0, gs1, ss0, ss1):
    c = lax.axis_index("c")
    s = lax.axis_index("s")
    sbufs = (sb0, sb1, sb2)
    dbufs = (db0, db1, db2)
    rbufs = (rb0, rb1)
    isems = (is0, is1, is2)
    gsems = (gs0, gs1)
    ssems = (ss0, ss1)
    base = s * EPT

    def aggregate(table, srcE, dstE, out, slot):
        pltpu.sync_copy(z2, acc.at[pl.ds(s * RPT, RPT)])
        plsc.subcore_barrier()

        def fire_idx(cn, k):
            pltpu.async_copy(srcE.at[pl.ds(base + cn * CH2, CH2)], sbufs[k], isems[k])
            pltpu.async_copy(dstE.at[pl.ds(base + cn * CH2, CH2)], dbufs[k], isems[k])

        def wait_idx(cn, k):
            pltpu.make_async_copy(srcE.at[pl.ds(base + cn * CH2, CH2)], sbufs[k], isems[k]).wait()
            pltpu.make_async_copy(dstE.at[pl.ds(base + cn * CH2, CH2)], dbufs[k], isems[k]).wait()

        def fire_gather(k, r):
            pltpu.async_copy(table.at[sbufs[k]], rbufs[r], gsems[r])

        def wait_gather(k, r):
            pltpu.make_async_copy(table.at[sbufs[k]], rbufs[r], gsems[r]).wait()

        def fire_scatter(k, r):
            pltpu.async_copy(rbufs[r], acc.at[dbufs[k]], ssems[r], add=True)

        def wait_scatter(k, r):
            pltpu.make_async_copy(rbufs[r], acc.at[dbufs[k]], ssems[r]).wait()

        def full_step(cn, u):
            wait_scatter((u - 1) % 3, (u - 1) % 2)
            fire_idx(cn + 2, (u + 2) % 3)
            wait_idx(cn + 1, (u + 1) % 3)
            fire_gather((u + 1) % 3, (u + 1) % 2)
            wait_gather(u % 3, u % 2)
            fire_scatter(u % 3, u % 2)

        fire_idx(0, 0)
        fire_idx(1, 1)
        wait_idx(0, 0)
        fire_gather(0, 0)
        fire_idx(2, 2)
        wait_idx(1, 1)
        fire_gather(1, 1)
        wait_gather(0, 0)
        fire_scatter(0, 0)

        def group(g, carry):
            cbase = 1 + 6 * g
            for uu in range(6):
                full_step(cbase + uu, (1 + uu) % 6)
            return carry

        lax.fori_loop(0, 12, group, 0)
        for cn in (73, 74, 75):
            full_step(cn, cn % 6)

        wait_scatter(75 % 3, 75 % 2)
        wait_idx(NCH2 - 1, (NCH2 - 1) % 3)
        fire_gather((NCH2 - 1) % 3, (NCH2 - 1) % 2)
        wait_gather(76 % 3, 76 % 2)
        fire_scatter(76 % 3, 76 % 2)
        wait_gather((NCH2 - 1) % 3, (NCH2 - 1) % 2)
        fire_scatter((NCH2 - 1) % 3, (NCH2 - 1) % 2)
        wait_scatter(76 % 3, 76 % 2)
        wait_scatter((NCH2 - 1) % 3, (NCH2 - 1) % 2)

        pltpu.sync_copy(srcE.at[pl.ds(base + NCH2 * CH2, TAIL)], tsb)
        pltpu.sync_copy(dstE.at[pl.ds(base + NCH2 * CH2, TAIL)], tdb)
        pltpu.async_copy(table.at[tsb], trb, is0).wait()
        pltpu.sync_copy(trb, acc.at[tdb], add=True)

        plsc.subcore_barrier()
        pltpu.sync_copy(acc.at[pl.ds(s * RPT, RPT)], out.at[slot, pl.ds(s * RPT, RPT)])
        plsc.subcore_barrier()

    @pl.when(c == 0)
    def _():
        aggregate(tab_lo, src1, dst1, out_lo, 1)
        aggregate(tab_lo, src2, dst2, out_lo, 0)

    @pl.when(c == 1)
    def _():
        aggregate(tab_hi, src1, dst1, out_hi, 1)
        aggregate(tab_hi, src2, dst2, out_hi, 0)


def _aggregate(tab_lo, tab_hi, src1, dst1, src2, dst2):
    z2 = jnp.zeros((RPT, DH), _f32)
    out = jax.ShapeDtypeStruct((2, NPAD, DH), _f32)
    idxbuf = pltpu.VMEM((CH2,), jnp.int32)
    fn = pl.kernel(
        _agg_body,
        out_type=[out, out],
        mesh=_sc_mesh(),
        scratch_types=[
            pltpu.VMEM_SHARED((NPAD, DH), _f32),
            idxbuf, idxbuf, idxbuf,
            idxbuf, idxbuf, idxbuf,
            pltpu.VMEM((CH2, DH), _f32),
            pltpu.VMEM((CH2, DH), _f32),
            pltpu.VMEM((TAIL,), jnp.int32),
            pltpu.VMEM((TAIL,), jnp.int32),
            pltpu.VMEM((TAIL, DH), _f32),
            pltpu.SemaphoreType.DMA,
            pltpu.SemaphoreType.DMA,
            pltpu.SemaphoreType.DMA,
            pltpu.SemaphoreType.DMA,
            pltpu.SemaphoreType.DMA,
            pltpu.SemaphoreType.DMA,
            pltpu.SemaphoreType.DMA,
        ],
    )
    return fn(tab_lo, tab_hi, src1, dst1, src2, dst2, z2)


BR = 1000


def _prep_body(x_ref, g_ref, b_ref, w_ref, deg_ref, proj_ref, lo_ref, hi_ref):
    x = x_ref[0]
    m = jnp.mean(x, axis=-1, keepdims=True)
    xc = x - m
    v = jnp.mean(xc * xc, axis=-1, keepdims=True)
    ln = xc * lax.rsqrt(v + 1e-5) * g_ref[0] + b_ref[0]
    proj_ref[0] = jnp.dot(ln.astype(_bf), w_ref[0].astype(_bf),
                          preferred_element_type=_f32)
    deg = deg_ref[0]
    ds = jnp.where(deg > 0, lax.rsqrt(jnp.where(deg > 0, deg, 1.0)), 0.0)
    xs = x * ds
    lo_ref[0] = xs[:, :DH]
    hi_ref[0] = xs[:, DH:]


def _prep(x2, g2, b2, w2, deg2):
    nb = N // BR
    return pl.pallas_call(
        _prep_body,
        grid=(2, nb),
        in_specs=[
            pl.BlockSpec((1, BR, D), lambda t, i: (t, i, 0)),
            pl.BlockSpec((1, 1, D), lambda t, i: (t, 0, 0)),
            pl.BlockSpec((1, 1, D), lambda t, i: (t, 0, 0)),
            pl.BlockSpec((1, D, D), lambda t, i: (t, 0, 0)),
            pl.BlockSpec((1, BR, 1), lambda t, i: (t, i, 0)),
        ],
        out_specs=[
            pl.BlockSpec((1, BR, D), lambda t, i: (t, i, 0)),
            pl.BlockSpec((1, BR, DH), lambda t, i: (t, i, 0)),
            pl.BlockSpec((1, BR, DH), lambda t, i: (t, i, 0)),
        ],
        out_shape=[
            jax.ShapeDtypeStruct((2, N, D), _f32),
            jax.ShapeDtypeStruct((2, N, DH), _f32),
            jax.ShapeDtypeStruct((2, N, DH), _f32),
        ],
    )(x2, g2, b2, w2, deg2)


def _erf(x):
    a1, a2, a3, a4, a5 = (0.254829592, -0.284496736, 1.421413741,
                          -1.453152027, 1.061405429)
    p = 0.3275911
    ax = jnp.abs(x)
    t = 1.0 / (1.0 + p * ax)
    poly = ((((a5 * t + a4) * t + a3) * t + a2) * t + a1) * t
    y = 1.0 - poly * jnp.exp(-ax * ax)
    return jnp.sign(x) * y


def _gelu(u):
    return 0.5 * u * (1.0 + _erf(u * 0.7071067811865476))


def _final_body(proj_ref, lo_ref, hi_ref, deg_ref, relW_ref, gate_ref,
                W1_ref, b1_ref, W2_ref, b2_ref, out_ref):
    proj = proj_ref[0]
    deg = deg_ref[0]
    ds = jnp.where(deg > 0, lax.rsqrt(jnp.where(deg > 0, deg, 1.0)), 0.0)
    gate = gate_ref[0, 0, 0]
    mlo = (lo_ref[0] * ds).astype(_bf)
    mhi = (hi_ref[0] * ds).astype(_bf)
    msg = (jnp.dot(mlo, relW_ref[0, :DH, :].astype(_bf), preferred_element_type=_f32)
           + jnp.dot(mhi, relW_ref[0, DH:, :].astype(_bf), preferred_element_type=_f32)) * gate
    u1 = (jnp.dot(proj.astype(_bf), W1_ref[0, :D, :].astype(_bf), preferred_element_type=_f32)
          + jnp.dot(msg.astype(_bf), W1_ref[0, D:, :].astype(_bf), preferred_element_type=_f32)
          + b1_ref[0])
    h = _gelu(u1)
    out_ref[0] = (proj
                  + jnp.dot(h.astype(_bf), W2_ref[0].astype(_bf),
                            preferred_element_type=_f32)
                  + b2_ref[0])


def _final(proj2, lo2, hi2, deg2, relW2, gate2, W12, b12, W22, b22):
    nb = N // BR
    return pl.pallas_call(
        _final_body,
        grid=(2, nb),
        in_specs=[
            pl.BlockSpec((1, BR, D), lambda t, i: (t, i, 0)),
            pl.BlockSpec((1, BR, DH), lambda t, i: (t, i, 0)),
            pl.BlockSpec((1, BR, DH), lambda t, i: (t, i, 0)),
            pl.BlockSpec((1, BR, 1), lambda t, i: (t, i, 0)),
            pl.BlockSpec((1, D, D), lambda t, i: (t, 0, 0)),
            pl.BlockSpec((1, 1, 1), lambda t, i: (t, 0, 0)),
            pl.BlockSpec((1, 2 * D, HID), lambda t, i: (t, 0, 0)),
            pl.BlockSpec((1, 1, HID), lambda t, i: (t, 0, 0)),
            pl.BlockSpec((1, HID, D), lambda t, i: (t, 0, 0)),
            pl.BlockSpec((1, 1, D), lambda t, i: (t, 0, 0)),
        ],
        out_specs=pl.BlockSpec((1, BR, D), lambda t, i: (t, i, 0)),
        out_shape=jax.ShapeDtypeStruct((2, N, D), _f32),
    )(proj2, lo2, hi2, deg2, relW2, gate2, W12, b12, W22, b22)


def kernel(x_A, x_B, ln_gA, ln_bA, resW_A, W1_A, b1_A, W2_A, b2_A,
           ln_gB, ln_bB, resW_B, W1_B, b1_B, W2_B, b2_B,
           relW_AB, gate_AB, relW_BA, gate_BA,
           edge_index_A_B, edge_index_B_A):
    e1 = edge_index_A_B.astype(jnp.int32)
    e2 = edge_index_B_A.astype(jnp.int32)
    src1_3d = e1[0].reshape(NT, NCH1, CH1)
    dst1_3d = e1[1].reshape(NT, NCH1, CH1)
    src2_3d = e2[0].reshape(NT, NCH1, CH1)
    dst2_3d = e2[1].reshape(NT, NCH1, CH1)
    srcF1 = e1[0]
    dstF1 = e1[1]
    srcF2 = e2[0] + N
    dstF2 = e2[1]

    degS, degD = _degrees(src1_3d, dst1_3d, src2_3d, dst2_3d)

    x2 = jnp.stack([x_A, x_B])
    g2 = jnp.stack([ln_gA, ln_gB]).reshape(2, 1, D)
    b2 = jnp.stack([ln_bA, ln_bB]).reshape(2, 1, D)
    w2 = jnp.stack([resW_A, resW_B])
    proj2, lo2, hi2 = _prep(x2, g2, b2, w2, degS.reshape(2, NPAD, 1)[:, :N])

    agg_lo, agg_hi = _aggregate(lo2.reshape(2 * N, DH), hi2.reshape(2 * N, DH),
                                srcF1, dstF1, srcF2, dstF2)

    relW2 = jnp.stack([relW_BA, relW_AB])
    gate2 = jnp.stack([gate_BA, gate_AB]).reshape(2, 1, 1)
    W12 = jnp.stack([W1_A, W1_B])
    b12 = jnp.stack([b1_A, b1_B]).reshape(2, 1, HID)
    W22 = jnp.stack([W2_A, W2_B])
    b22 = jnp.stack([b2_A, b2_B]).reshape(2, 1, D)
    out2 = _final(proj2, agg_lo, agg_hi, degD.reshape(2, NPAD, 1),
                  relW2, gate2, W12, b12, W22, b22)
    return out2.reshape(2 * N, D)


# speedup vs baseline: 4.9900x; 1.0003x over previous
import jax
import jax.numpy as jnp
from jax import lax
from jax.experimental import pallas as pl
from jax.experimental.pallas import tpu as pltpu
from jax.experimental.pallas import tpu_sc as plsc

N = 10000
E = 160000
D = 256
DH = 128
HID = 512
NT = 16
EPT = E // NT
NPAD = 10240
RPT = NPAD // NT

CH1 = 80
NCH1 = EPT // CH1
KB = 5

CH2 = 128
NCH2 = EPT // CH2
TAIL = EPT - NCH2 * CH2

_f32 = jnp.float32
_bf = jnp.bfloat16


def _sc_mesh():
    return plsc.VectorSubcoreMesh(core_axis_name="c", subcore_axis_name="s")


def _deg_body(src1, dst1, src2, dst2, z1,
              degS, degD,
              accS, accD, srcv, dstv, ones_v, semS, semD):
    c = lax.axis_index("c")
    s = lax.axis_index("s")
    for i in range(CH1 // 16):
        ones_v[pl.ds(i * 16, 16)] = jnp.ones((16,), _f32)
    pltpu.sync_copy(z1, accS.at[pl.ds(s * RPT, RPT)])
    pltpu.sync_copy(z1, accD.at[pl.ds(s * RPT, RPT)])
    plsc.subcore_barrier()

    def accum(src3d, dst3d):
        pltpu.sync_copy(src3d.at[s], srcv)
        pltpu.sync_copy(dst3d.at[s], dstv)

        def group(g, carry):
            for u in range(KB):
                j = KB * g + u
                pltpu.async_copy(ones_v, accS.at[srcv.at[j]], semS, add=True)
                pltpu.async_copy(ones_v, accD.at[dstv.at[j]], semD, add=True)
            for u in range(KB):
                j = KB * g + u
                pltpu.make_async_copy(ones_v, accS.at[srcv.at[j]], semS).wait()
                pltpu.make_async_copy(ones_v, accD.at[dstv.at[j]], semD).wait()
            return carry

        lax.fori_loop(0, NCH1 // KB, group, 0)

    @pl.when(c == 0)
    def _():
        accum(src1, dst1)

    @pl.when(c == 1)
    def _():
        accum(src2, dst2)

    plsc.subcore_barrier()
    @pl.when(c == 0)
    def _():
        pltpu.sync_copy(accS.at[pl.ds(s * RPT, RPT)], degS.at[0, pl.ds(s * RPT, RPT)])
        pltpu.sync_copy(accD.at[pl.ds(s * RPT, RPT)], degD.at[1, pl.ds(s * RPT, RPT)])

    @pl.when(c == 1)
    def _():
        pltpu.sync_copy(accS.at[pl.ds(s * RPT, RPT)], degS.at[1, pl.ds(s * RPT, RPT)])
        pltpu.sync_copy(accD.at[pl.ds(s * RPT, RPT)], degD.at[0, pl.ds(s * RPT, RPT)])


def _degrees(src1, dst1, src2, dst2):
    z1 = jnp.zeros((RPT,), _f32)
    out = jax.ShapeDtypeStruct((2, NPAD), _f32)
    fn = pl.kernel(
        _deg_body,
        out_type=[out, out],
        mesh=_sc_mesh(),
        scratch_types=[
            pltpu.VMEM_SHARED((NPAD,), _f32),
            pltpu.VMEM_SHARED((NPAD,), _f32),
            pltpu.VMEM((NCH1, CH1), jnp.int32),
            pltpu.VMEM((NCH1, CH1), jnp.int32),
            pltpu.VMEM((CH1,), _f32),
            pltpu.SemaphoreType.DMA,
            pltpu.SemaphoreType.DMA,
        ],
    )
    return fn(src1, dst1, src2, dst2, z1)


def _agg_body(tab_lo, tab_hi, src1, dst1, src2, dst2, z2,
              out_lo, out_hi,
              acc, sb0, sb1, sb2, db0, db1, db2, rb0, rb1, tsb, tdb, trb,
              is0, is1, is2, gs0, gs1, ss0, ss1):
    c = lax.axis_index("c")
    s = lax.axis_index("s")
    sbufs = (sb0, sb1, sb2)
    dbufs = (db0, db1, db2)
    rbufs = (rb0, rb1)
    isems = (is0, is1, is2)
    gsems = (gs0, gs1)
    ssems = (ss0, ss1)
    base = s * EPT

    def aggregate(table, srcE, dstE, out, slot):
        pltpu.sync_copy(z2, acc.at[pl.ds(s * RPT, RPT)])
        plsc.subcore_barrier()

        def fire_idx(cn, k):
            pltpu.async_copy(srcE.at[pl.ds(base + cn * CH2, CH2)], sbufs[k], isems[k])
            pltpu.async_copy(dstE.at[pl.ds(base + cn * CH2, CH2)], dbufs[k], isems[k])

        def wait_idx(cn, k):
            pltpu.make_async_copy(srcE.at[pl.ds(base + cn * CH2, CH2)], sbufs[k], isems[k]).wait()
            pltpu.make_async_copy(dstE.at[pl.ds(base + cn * CH2, CH2)], dbufs[k], isems[k]).wait()

        def fire_gather(k, r):
            pltpu.async_copy(table.at[sbufs[k]], rbufs[r], gsems[r])

        def wait_gather(k, r):
            pltpu.make_async_copy(table.at[sbufs[k]], rbufs[r], gsems[r]).wait()

        def fire_scatter(k, r):
            pltpu.async_copy(rbufs[r], acc.at[dbufs[k]], ssems[r], add=True)

        def wait_scatter(k, r):
            pltpu.make_async_copy(rbufs[r], acc.at[dbufs[k]], ssems[r]).wait()

        def full_step(cn, u):
            wait_scatter((u - 1) % 3, (u - 1) % 2)
            fire_idx(cn + 2, (u + 2) % 3)
            wait_idx(cn + 1, (u + 1) % 3)
            fire_gather((u + 1) % 3, (u + 1) % 2)
            wait_gather(u % 3, u % 2)
            fire_scatter(u % 3, u % 2)

        fire_idx(0, 0)
        fire_idx(1, 1)
        wait_idx(0, 0)
        fire_gather(0, 0)
        fire_idx(2, 2)
        wait_idx(1, 1)
        fire_gather(1, 1)
        wait_gather(0, 0)
        fire_scatter(0, 0)

        def group(g, carry):
            cbase = 1 + 6 * g
            for uu in range(6):
                full_step(cbase + uu, (1 + uu) % 6)
            return carry

        lax.fori_loop(0, 12, group, 0)
        for cn in (73, 74, 75):
            full_step(cn, cn % 6)

        wait_scatter(75 % 3, 75 % 2)
        wait_idx(NCH2 - 1, (NCH2 - 1) % 3)
        fire_gather((NCH2 - 1) % 3, (NCH2 - 1) % 2)
        wait_gather(76 % 3, 76 % 2)
        fire_scatter(76 % 3, 76 % 2)
        wait_gather((NCH2 - 1) % 3, (NCH2 - 1) % 2)
        fire_scatter((NCH2 - 1) % 3, (NCH2 - 1) % 2)
        wait_scatter(76 % 3, 76 % 2)
        wait_scatter((NCH2 - 1) % 3, (NCH2 - 1) % 2)

        pltpu.sync_copy(srcE.at[pl.ds(base + NCH2 * CH2, TAIL)], tsb)
        pltpu.sync_copy(dstE.at[pl.ds(base + NCH2 * CH2, TAIL)], tdb)
        pltpu.async_copy(table.at[tsb], trb, is0).wait()
        pltpu.sync_copy(trb, acc.at[tdb], add=True)

        plsc.subcore_barrier()
        pltpu.sync_copy(acc.at[pl.ds(s * RPT, RPT)], out.at[slot, pl.ds(s * RPT, RPT)])
        plsc.subcore_barrier()

    @pl.when(c == 0)
    def _():
        aggregate(tab_lo, src1, dst1, out_lo, 1)
        aggregate(tab_lo, src2, dst2, out_lo, 0)

    @pl.when(c == 1)
    def _():
        aggregate(tab_hi, src1, dst1, out_hi, 1)
        aggregate(tab_hi, src2, dst2, out_hi, 0)


def _aggregate(tab_lo, tab_hi, src1, dst1, src2, dst2):
    z2 = jnp.zeros((RPT, DH), _f32)
    out = jax.ShapeDtypeStruct((2, NPAD, DH), _f32)
    idxbuf = pltpu.VMEM((CH2,), jnp.int32)
    fn = pl.kernel(
        _agg_body,
        out_type=[out, out],
        mesh=_sc_mesh(),
        scratch_types=[
            pltpu.VMEM_SHARED((NPAD, DH), _f32),
            idxbuf, idxbuf, idxbuf,
            idxbuf, idxbuf, idxbuf,
            pltpu.VMEM((CH2, DH), _f32),
            pltpu.VMEM((CH2, DH), _f32),
            pltpu.VMEM((TAIL,), jnp.int32),
            pltpu.VMEM((TAIL,), jnp.int32),
            pltpu.VMEM((TAIL, DH), _f32),
            pltpu.SemaphoreType.DMA,
            pltpu.SemaphoreType.DMA,
            pltpu.SemaphoreType.DMA,
            pltpu.SemaphoreType.DMA,
            pltpu.SemaphoreType.DMA,
            pltpu.SemaphoreType.DMA,
            pltpu.SemaphoreType.DMA,
        ],
    )
    return fn(tab_lo, tab_hi, src1, dst1, src2, dst2, z2)


BR = 1000


def _prep_body(x_ref, g_ref, b_ref, w_ref, deg_ref, proj_ref, lo_ref, hi_ref):
    x = x_ref[0]
    m = jnp.mean(x, axis=-1, keepdims=True)
    xc = x - m
    v = jnp.mean(xc * xc, axis=-1, keepdims=True)
    ln = xc * lax.rsqrt(v + 1e-5) * g_ref[0] + b_ref[0]
    proj_ref[0] = jnp.dot(ln.astype(_bf), w_ref[0].astype(_bf),
                          preferred_element_type=_f32)
    deg = deg_ref[0]
    ds = jnp.where(deg > 0, lax.rsqrt(jnp.where(deg > 0, deg, 1.0)), 0.0)
    xs = x * ds
    lo_ref[0] = xs[:, :DH]
    hi_ref[0] = xs[:, DH:]


def _prep(x2, g2, b2, w2, deg2):
    nb = N // BR
    return pl.pallas_call(
        _prep_body,
        grid=(2, nb),
        in_specs=[
            pl.BlockSpec((1, BR, D), lambda t, i: (t, i, 0)),
            pl.BlockSpec((1, 1, D), lambda t, i: (t, 0, 0)),
            pl.BlockSpec((1, 1, D), lambda t, i: (t, 0, 0)),
            pl.BlockSpec((1, D, D), lambda t, i: (t, 0, 0)),
            pl.BlockSpec((1, BR, 1), lambda t, i: (t, i, 0)),
        ],
        out_specs=[
            pl.BlockSpec((1, BR, D), lambda t, i: (t, i, 0)),
            pl.BlockSpec((1, BR, DH), lambda t, i: (t, i, 0)),
            pl.BlockSpec((1, BR, DH), lambda t, i: (t, i, 0)),
        ],
        out_shape=[
            jax.ShapeDtypeStruct((2, N, D), _f32),
            jax.ShapeDtypeStruct((2, N, DH), _f32),
            jax.ShapeDtypeStruct((2, N, DH), _f32),
        ],
    )(x2, g2, b2, w2, deg2)


def _erf(x):
    a1, a2, a3, a4, a5 = (0.254829592, -0.284496736, 1.421413741,
                          -1.453152027, 1.061405429)
    p = 0.3275911
    ax = jnp.abs(x)
    t = 1.0 / (1.0 + p * ax)
    poly = ((((a5 * t + a4) * t + a3) * t + a2) * t + a1) * t
    y = 1.0 - poly * jnp.exp(-ax * ax)
    return jnp.sign(x) * y


def _gelu(u):
    return 0.5 * u * (1.0 + _erf(u * 0.7071067811865476))


def _final_body(proj_ref, lo_ref, hi_ref, deg_ref, relW_ref, gate_ref,
                W1_ref, b1_ref, W2_ref, b2_ref, out_ref):
    proj = proj_ref[0]
    deg = deg_ref[0]
    ds = jnp.where(deg > 0, lax.rsqrt(jnp.where(deg > 0, deg, 1.0)), 0.0)
    gate = gate_ref[0, 0, 0]
    mlo = (lo_ref[0] * ds).astype(_bf)
    mhi = (hi_ref[0] * ds).astype(_bf)
    msg = (jnp.dot(mlo, relW_ref[0, :DH, :].astype(_bf), preferred_element_type=_f32)
           + jnp.dot(mhi, relW_ref[0, DH:, :].astype(_bf), preferred_element_type=_f32)) * gate
    u1 = (jnp.dot(proj.astype(_bf), W1_ref[0, :D, :].astype(_bf), preferred_element_type=_f32)
          + jnp.dot(msg.astype(_bf), W1_ref[0, D:, :].astype(_bf), preferred_element_type=_f32)
          + b1_ref[0])
    h = _gelu(u1)
    out_ref[0] = (proj
                  + jnp.dot(h.astype(_bf), W2_ref[0].astype(_bf),
                            preferred_element_type=_f32)
                  + b2_ref[0])


def _final(proj2, lo2, hi2, deg2, relW2, gate2, W12, b12, W22, b22):
    nb = N // BR
    return pl.pallas_call(
        _final_body,
        grid=(2, nb),
        in_specs=[
            pl.BlockSpec((1, BR, D), lambda t, i: (t, i, 0)),
            pl.BlockSpec((1, BR, DH), lambda t, i: (t, i, 0)),
            pl.BlockSpec((1, BR, DH), lambda t, i: (t, i, 0)),
            pl.BlockSpec((1, BR, 1), lambda t, i: (t, i, 0)),
            pl.BlockSpec((1, D, D), lambda t, i: (t, 0, 0)),
            pl.BlockSpec((1, 1, 1), lambda t, i: (t, 0, 0)),
            pl.BlockSpec((1, 2 * D, HID), lambda t, i: (t, 0, 0)),
            pl.BlockSpec((1, 1, HID), lambda t, i: (t, 0, 0)),
            pl.BlockSpec((1, HID, D), lambda t, i: (t, 0, 0)),
            pl.BlockSpec((1, 1, D), lambda t, i: (t, 0, 0)),
        ],
        out_specs=pl.BlockSpec((1, BR, D), lambda t, i: (t, i, 0)),
        out_shape=jax.ShapeDtypeStruct((2, N, D), _f32),
    )(proj2, lo2, hi2, deg2, relW2, gate2, W12, b12, W22, b22)


def kernel(x_A, x_B, ln_gA, ln_bA, resW_A, W1_A, b1_A, W2_A, b2_A,
           ln_gB, ln_bB, resW_B, W1_B, b1_B, W2_B, b2_B,
           relW_AB, gate_AB, relW_BA, gate_BA,
           edge_index_A_B, edge_index_B_A):
    e1 = edge_index_A_B.astype(jnp.int32)
    e2 = edge_index_B_A.astype(jnp.int32)
    src1_3d = e1[0].reshape(NT, NCH1, CH1)
    dst1_3d = e1[1].reshape(NT, NCH1, CH1)
    src2_3d = e2[0].reshape(NT, NCH1, CH1)
    dst2_3d = e2[1].reshape(NT, NCH1, CH1)
    srcF1 = e1[0]
    dstF1 = e1[1]
    srcF2 = e2[0] + N
    dstF2 = e2[1]

    degS, degD = _degrees(src1_3d, dst1_3d, src2_3d, dst2_3d)

    x2 = jnp.stack([x_A, x_B])
    g2 = jnp.stack([ln_gA, ln_gB]).reshape(2, 1, D)
    b2 = jnp.stack([ln_bA, ln_bB]).reshape(2, 1, D)
    w2 = jnp.stack([resW_A, resW_B])
    proj2, lo2, hi2 = _prep(x2, g2, b2, w2, degS.reshape(2, NPAD, 1)[:, :N])

    agg_lo, agg_hi = _aggregate(lo2.reshape(2 * N, DH), hi2.reshape(2 * N, DH),
                                srcF1, dstF1, srcF2, dstF2)

    relW2 = jnp.stack([relW_BA, relW_AB])
    gate2 = jnp.stack([gate_BA, gate_AB]).reshape(2, 1, 1)
    W12 = jnp.stack([W1_A, W1_B])
    b12 = jnp.stack([b1_A, b1_B]).reshape(2, 1, HID)
    W22 = jnp.stack([W2_A, W2_B])
    b22 = jnp.stack([b2_A, b2_B]).reshape(2, 1, D)
    out2 = _final(proj2, agg_lo, agg_hi, degD.reshape(2, NPAD, 1),
                  relW2, gate2, W12, b12, W22, b22)
    return out2.reshape(2 * N, D)


# speedup vs baseline: 4.9922x; 1.0004x over previous
import jax
import jax.numpy as jnp
from jax import lax
from jax.experimental import pallas as pl
from jax.experimental.pallas import tpu as pltpu
from jax.experimental.pallas import tpu_sc as plsc

N = 10000
E = 160000
D = 256
DH = 128
HID = 512
NT = 16
EPT = E // NT
NPAD = 10240
RPT = NPAD // NT

CH1 = 128
NCH1 = EPT // CH1
TL1 = EPT - NCH1 * CH1
KB = 6

CH2 = 128
NCH2 = EPT // CH2
TAIL = EPT - NCH2 * CH2

_f32 = jnp.float32
_bf = jnp.bfloat16


def _sc_mesh():
    return plsc.VectorSubcoreMesh(core_axis_name="c", subcore_axis_name="s")


def _deg_body(src1, dst1, src2, dst2, st1, dt1, st2, dt2, z1,
              degS, degD,
              accS, accD, srcv, dstv, stv, dtv, ones_v, semS, semD):
    c = lax.axis_index("c")
    s = lax.axis_index("s")
    for i in range(CH1 // 16):
        ones_v[pl.ds(i * 16, 16)] = jnp.ones((16,), _f32)
    pltpu.sync_copy(z1, accS.at[pl.ds(s * RPT, RPT)])
    pltpu.sync_copy(z1, accD.at[pl.ds(s * RPT, RPT)])
    plsc.subcore_barrier()

    def accum(src3d, dst3d, srct, dstt):
        pltpu.sync_copy(src3d.at[s], srcv)
        pltpu.sync_copy(dst3d.at[s], dstv)
        pltpu.sync_copy(srct.at[s], stv)
        pltpu.sync_copy(dstt.at[s], dtv)

        def group(g, carry):
            for u in range(KB):
                j = KB * g + u
                pltpu.async_copy(ones_v, accS.at[srcv.at[j]], semS, add=True)
                pltpu.async_copy(ones_v, accD.at[dstv.at[j]], semD, add=True)
            for u in range(KB):
                j = KB * g + u
                pltpu.make_async_copy(ones_v, accS.at[srcv.at[j]], semS).wait()
                pltpu.make_async_copy(ones_v, accD.at[dstv.at[j]], semD).wait()
            return carry

        lax.fori_loop(0, NCH1 // KB, group, 0)
        pltpu.sync_copy(ones_v.at[pl.ds(0, TL1)], accS.at[stv], add=True)
        pltpu.sync_copy(ones_v.at[pl.ds(0, TL1)], accD.at[dtv], add=True)

    @pl.when(c == 0)
    def _():
        accum(src1, dst1, st1, dt1)

    @pl.when(c == 1)
    def _():
        accum(src2, dst2, st2, dt2)

    plsc.subcore_barrier()
    @pl.when(c == 0)
    def _():
        pltpu.sync_copy(accS.at[pl.ds(s * RPT, RPT)], degS.at[0, pl.ds(s * RPT, RPT)])
        pltpu.sync_copy(accD.at[pl.ds(s * RPT, RPT)], degD.at[1, pl.ds(s * RPT, RPT)])

    @pl.when(c == 1)
    def _():
        pltpu.sync_copy(accS.at[pl.ds(s * RPT, RPT)], degS.at[1, pl.ds(s * RPT, RPT)])
        pltpu.sync_copy(accD.at[pl.ds(s * RPT, RPT)], degD.at[0, pl.ds(s * RPT, RPT)])


def _degrees(src1, dst1, src2, dst2, st1, dt1, st2, dt2):
    z1 = jnp.zeros((RPT,), _f32)
    out = jax.ShapeDtypeStruct((2, NPAD), _f32)
    fn = pl.kernel(
        _deg_body,
        out_type=[out, out],
        mesh=_sc_mesh(),
        scratch_types=[
            pltpu.VMEM_SHARED((NPAD,), _f32),
            pltpu.VMEM_SHARED((NPAD,), _f32),
            pltpu.VMEM((NCH1, CH1), jnp.int32),
            pltpu.VMEM((NCH1, CH1), jnp.int32),
            pltpu.VMEM((TL1,), jnp.int32),
            pltpu.VMEM((TL1,), jnp.int32),
            pltpu.VMEM((CH1,), _f32),
            pltpu.SemaphoreType.DMA,
            pltpu.SemaphoreType.DMA,
        ],
    )
    return fn(src1, dst1, src2, dst2, st1, dt1, st2, dt2, z1)


def _agg_body(tab_lo, tab_hi, src1, dst1, src2, dst2, z2,
              out_lo, out_hi,
              acc, sb0, sb1, sb2, db0, db1, db2, rb0, rb1, tsb, tdb, trb,
              is0, is1, is2, gs0, gs1, ss0, ss1):
    c = lax.axis_index("c")
    s = lax.axis_index("s")
    sbufs = (sb0, sb1, sb2)
    dbufs = (db0, db1, db2)
    rbufs = (rb0, rb1)
    isems = (is0, is1, is2)
    gsems = (gs0, gs1)
    ssems = (ss0, ss1)
    base = s * EPT

    def aggregate(table, srcE, dstE, out, slot):
        pltpu.sync_copy(z2, acc.at[pl.ds(s * RPT, RPT)])
        plsc.subcore_barrier()

        def fire_idx(cn, k):
            pltpu.async_copy(srcE.at[pl.ds(base + cn * CH2, CH2)], sbufs[k], isems[k])
            pltpu.async_copy(dstE.at[pl.ds(base + cn * CH2, CH2)], dbufs[k], isems[k])

        def wait_idx(cn, k):
            pltpu.make_async_copy(srcE.at[pl.ds(base + cn * CH2, CH2)], sbufs[k], isems[k]).wait()
            pltpu.make_async_copy(dstE.at[pl.ds(base + cn * CH2, CH2)], dbufs[k], isems[k]).wait()

        def fire_gather(k, r):
            pltpu.async_copy(table.at[sbufs[k]], rbufs[r], gsems[r])

        def wait_gather(k, r):
            pltpu.make_async_copy(table.at[sbufs[k]], rbufs[r], gsems[r]).wait()

        def fire_scatter(k, r):
            pltpu.async_copy(rbufs[r], acc.at[dbufs[k]], ssems[r], add=True)

        def wait_scatter(k, r):
            pltpu.make_async_copy(rbufs[r], acc.at[dbufs[k]], ssems[r]).wait()

        def full_step(cn, u):
            wait_scatter((u - 1) % 3, (u - 1) % 2)
            fire_idx(cn + 2, (u + 2) % 3)
            wait_idx(cn + 1, (u + 1) % 3)
            fire_gather((u + 1) % 3, (u + 1) % 2)
            wait_gather(u % 3, u % 2)
            fire_scatter(u % 3, u % 2)

        fire_idx(0, 0)
        fire_idx(1, 1)
        wait_idx(0, 0)
        fire_gather(0, 0)
        fire_idx(2, 2)
        wait_idx(1, 1)
        fire_gather(1, 1)
        wait_gather(0, 0)
        fire_scatter(0, 0)

        def group(g, carry):
            cbase = 1 + 6 * g
            for uu in range(6):
                full_step(cbase + uu, (1 + uu) % 6)
            return carry

        lax.fori_loop(0, 12, group, 0)
        for cn in (73, 74, 75):
            full_step(cn, cn % 6)

        wait_scatter(75 % 3, 75 % 2)
        wait_idx(NCH2 - 1, (NCH2 - 1) % 3)
        fire_gather((NCH2 - 1) % 3, (NCH2 - 1) % 2)
        wait_gather(76 % 3, 76 % 2)
        fire_scatter(76 % 3, 76 % 2)
        wait_gather((NCH2 - 1) % 3, (NCH2 - 1) % 2)
        fire_scatter((NCH2 - 1) % 3, (NCH2 - 1) % 2)
        wait_scatter(76 % 3, 76 % 2)
        wait_scatter((NCH2 - 1) % 3, (NCH2 - 1) % 2)

        pltpu.sync_copy(srcE.at[pl.ds(base + NCH2 * CH2, TAIL)], tsb)
        pltpu.sync_copy(dstE.at[pl.ds(base + NCH2 * CH2, TAIL)], tdb)
        pltpu.async_copy(table.at[tsb], trb, is0).wait()
        pltpu.sync_copy(trb, acc.at[tdb], add=True)

        plsc.subcore_barrier()
        pltpu.sync_copy(acc.at[pl.ds(s * RPT, RPT)], out.at[slot, pl.ds(s * RPT, RPT)])
        plsc.subcore_barrier()

    @pl.when(c == 0)
    def _():
        aggregate(tab_lo, src1, dst1, out_lo, 1)
        aggregate(tab_lo, src2, dst2, out_lo, 0)

    @pl.when(c == 1)
    def _():
        aggregate(tab_hi, src1, dst1, out_hi, 1)
        aggregate(tab_hi, src2, dst2, out_hi, 0)


def _aggregate(tab_lo, tab_hi, src1, dst1, src2, dst2):
    z2 = jnp.zeros((RPT, DH), _f32)
    out = jax.ShapeDtypeStruct((2, NPAD, DH), _f32)
    idxbuf = pltpu.VMEM((CH2,), jnp.int32)
    fn = pl.kernel(
        _agg_body,
        out_type=[out, out],
        mesh=_sc_mesh(),
        scratch_types=[
            pltpu.VMEM_SHARED((NPAD, DH), _f32),
            idxbuf, idxbuf, idxbuf,
            idxbuf, idxbuf, idxbuf,
            pltpu.VMEM((CH2, DH), _f32),
            pltpu.VMEM((CH2, DH), _f32),
            pltpu.VMEM((TAIL,), jnp.int32),
            pltpu.VMEM((TAIL,), jnp.int32),
            pltpu.VMEM((TAIL, DH), _f32),
            pltpu.SemaphoreType.DMA,
            pltpu.SemaphoreType.DMA,
            pltpu.SemaphoreType.DMA,
            pltpu.SemaphoreType.DMA,
            pltpu.SemaphoreType.DMA,
            pltpu.SemaphoreType.DMA,
            pltpu.SemaphoreType.DMA,
        ],
    )
    return fn(tab_lo, tab_hi, src1, dst1, src2, dst2, z2)


BR = 1000


def _prep_body(xA_ref, xB_ref, g_ref, b_ref, w_ref, deg_ref, proj_ref, lo_ref, hi_ref):
    t = pl.program_id(0)
    x = jnp.where(t == 0, xA_ref[...], xB_ref[...])
    m = jnp.mean(x, axis=-1, keepdims=True)
    xc = x - m
    v = jnp.mean(xc * xc, axis=-1, keepdims=True)
    ln = xc * lax.rsqrt(v + 1e-5) * g_ref[0] + b_ref[0]
    proj_ref[0] = jnp.dot(ln.astype(_bf), w_ref[0].astype(_bf),
                          preferred_element_type=_f32)
    deg = deg_ref[0]
    ds = jnp.where(deg > 0, lax.rsqrt(jnp.where(deg > 0, deg, 1.0)), 0.0)
    xs = x * ds
    lo_ref[0] = xs[:, :DH]
    hi_ref[0] = xs[:, DH:]


def _prep(xA, xB, g2, b2, w2, deg2):
    nb = N // BR
    return pl.pallas_call(
        _prep_body,
        grid=(2, nb),
        in_specs=[
            pl.BlockSpec((BR, D), lambda t, i: (i, 0)),
            pl.BlockSpec((BR, D), lambda t, i: (i, 0)),
            pl.BlockSpec((1, 1, D), lambda t, i: (t, 0, 0)),
            pl.BlockSpec((1, 1, D), lambda t, i: (t, 0, 0)),
            pl.BlockSpec((1, D, D), lambda t, i: (t, 0, 0)),
            pl.BlockSpec((1, BR, 1), lambda t, i: (t, i, 0)),
        ],
        out_specs=[
            pl.BlockSpec((1, BR, D), lambda t, i: (t, i, 0)),
            pl.BlockSpec((1, BR, DH), lambda t, i: (t, i, 0)),
            pl.BlockSpec((1, BR, DH), lambda t, i: (t, i, 0)),
        ],
        out_shape=[
            jax.ShapeDtypeStruct((2, N, D), _f32),
            jax.ShapeDtypeStruct((2, N, DH), _f32),
            jax.ShapeDtypeStruct((2, N, DH), _f32),
        ],
    )(xA, xB, g2, b2, w2, deg2)


def _erf(x):
    a1, a2, a3, a4, a5 = (0.254829592, -0.284496736, 1.421413741,
                          -1.453152027, 1.061405429)
    p = 0.3275911
    ax = jnp.abs(x)
    t = 1.0 / (1.0 + p * ax)
    poly = ((((a5 * t + a4) * t + a3) * t + a2) * t + a1) * t
    y = 1.0 - poly * jnp.exp(-ax * ax)
    return jnp.sign(x) * y


def _gelu(u):
    return 0.5 * u * (1.0 + _erf(u * 0.7071067811865476))


def _final_body(proj_ref, lo_ref, hi_ref, deg_ref, relW_ref, gate_ref,
                W1_ref, b1_ref, W2_ref, b2_ref, out_ref):
    proj = proj_ref[0]
    deg = deg_ref[0]
    ds = jnp.where(deg > 0, lax.rsqrt(jnp.where(deg > 0, deg, 1.0)), 0.0)
    gate = gate_ref[0, 0, 0]
    mlo = (lo_ref[0] * ds).astype(_bf)
    mhi = (hi_ref[0] * ds).astype(_bf)
    msg = (jnp.dot(mlo, relW_ref[0, :DH, :].astype(_bf), preferred_element_type=_f32)
           + jnp.dot(mhi, relW_ref[0, DH:, :].astype(_bf), preferred_element_type=_f32)) * gate
    u1 = (jnp.dot(proj.astype(_bf), W1_ref[0, :D, :].astype(_bf), preferred_element_type=_f32)
          + jnp.dot(msg.astype(_bf), W1_ref[0, D:, :].astype(_bf), preferred_element_type=_f32)
          + b1_ref[0])
    h = _gelu(u1)
    out_ref[0] = (proj
                  + jnp.dot(h.astype(_bf), W2_ref[0].astype(_bf),
                            preferred_element_type=_f32)
                  + b2_ref[0])


def _final(proj2, lo2, hi2, deg2, relW2, gate2, W12, b12, W22, b22):
    nb = N // BR
    return pl.pallas_call(
        _final_body,
        grid=(2, nb),
        in_specs=[
            pl.BlockSpec((1, BR, D), lambda t, i: (t, i, 0)),
            pl.BlockSpec((1, BR, DH), lambda t, i: (t, i, 0)),
            pl.BlockSpec((1, BR, DH), lambda t, i: (t, i, 0)),
            pl.BlockSpec((1, BR, 1), lambda t, i: (t, i, 0)),
            pl.BlockSpec((1, D, D), lambda t, i: (t, 0, 0)),
            pl.BlockSpec((1, 1, 1), lambda t, i: (t, 0, 0)),
            pl.BlockSpec((1, 2 * D, HID), lambda t, i: (t, 0, 0)),
            pl.BlockSpec((1, 1, HID), lambda t, i: (t, 0, 0)),
            pl.BlockSpec((1, HID, D), lambda t, i: (t, 0, 0)),
            pl.BlockSpec((1, 1, D), lambda t, i: (t, 0, 0)),
        ],
        out_specs=pl.BlockSpec((1, BR, D), lambda t, i: (t, i, 0)),
        out_shape=jax.ShapeDtypeStruct((2, N, D), _f32),
    )(proj2, lo2, hi2, deg2, relW2, gate2, W12, b12, W22, b22)


def kernel(x_A, x_B, ln_gA, ln_bA, resW_A, W1_A, b1_A, W2_A, b2_A,
           ln_gB, ln_bB, resW_B, W1_B, b1_B, W2_B, b2_B,
           relW_AB, gate_AB, relW_BA, gate_BA,
           edge_index_A_B, edge_index_B_A):
    e1 = edge_index_A_B.astype(jnp.int32)
    e2 = edge_index_B_A.astype(jnp.int32)

    def split_main_tail(v):
        v2 = v.reshape(NT, EPT)
        return (v2[:, :NCH1 * CH1].reshape(NT, NCH1, CH1), v2[:, NCH1 * CH1:])

    src1_3d, st1 = split_main_tail(e1[0])
    dst1_3d, dt1 = split_main_tail(e1[1])
    src2_3d, st2 = split_main_tail(e2[0])
    dst2_3d, dt2 = split_main_tail(e2[1])
    srcF1 = e1[0]
    dstF1 = e1[1]
    srcF2 = e2[0] + N
    dstF2 = e2[1]

    degS, degD = _degrees(src1_3d, dst1_3d, src2_3d, dst2_3d,
                          st1, dt1, st2, dt2)

    g2 = jnp.stack([ln_gA, ln_gB]).reshape(2, 1, D)
    b2 = jnp.stack([ln_bA, ln_bB]).reshape(2, 1, D)
    w2 = jnp.stack([resW_A, resW_B])
    proj2, lo2, hi2 = _prep(x_A, x_B, g2, b2, w2,
                            degS.reshape(2, NPAD, 1)[:, :N])

    agg_lo, agg_hi = _aggregate(lo2.reshape(2 * N, DH), hi2.reshape(2 * N, DH),
                                srcF1, dstF1, srcF2, dstF2)

    relW2 = jnp.stack([relW_BA, relW_AB])
    gate2 = jnp.stack([gate_BA, gate_AB]).reshape(2, 1, 1)
    W12 = jnp.stack([W1_A, W1_B])
    b12 = jnp.stack([b1_A, b1_B]).reshape(2, 1, HID)
    W22 = jnp.stack([W2_A, W2_B])
    b22 = jnp.stack([b2_A, b2_B]).reshape(2, 1, D)
    out2 = _final(proj2, agg_lo, agg_hi, degD.reshape(2, NPAD, 1),
                  relW2, gate2, W12, b12, W22, b22)
    return out2.reshape(2 * N, D)
